# Initial kernel scaffold; baseline (speedup 1.0000x reference)
import jax
import jax.numpy as jnp
from jax.experimental import pallas as pl


def kernel(pc1, mask, W1, b1, W2, b2, W3, b3):
    raise NotImplementedError("write your pallas kernel here")


# speedup vs baseline: 1.6046x; 1.6046x over previous
import jax
import jax.numpy as jnp
from jax.experimental import pallas as pl
from jax.experimental.pallas import tpu as pltpu

_N = 100000
_I = 10
_H = 128
_BN = 2000
_NEG = -3.0e38
_POS = 3.0e38
_INTERPRET = False


def _stage1_body(pc_ref, mk_ref, w1_ref, b1_ref, w2_ref, b2_ref, w3_ref, b3_ref,
                 m_ref, t_ref, yaw_ref, ind_ref, vmax_ref, vmin_ref,
                 smax_ref, smin_ref):
    i = pl.program_id(0)
    nb = pl.num_programs(0)
    x = pc_ref[...]
    mk = mk_ref[...]

    h = jnp.maximum(
        jnp.dot(x, w1_ref[...], preferred_element_type=jnp.float32)
        + b1_ref[...][None, :], 0.0)
    h = jnp.maximum(
        jnp.dot(h, w2_ref[...], preferred_element_type=jnp.float32)
        + b2_ref[...][None, :], 0.0)
    out4 = (jnp.dot(h, w3_ref[...], preferred_element_type=jnp.float32)
            + b3_ref[...][None, :])
    t_ref[...] = out4[:, :3]
    yaw_ref[...] = out4[:, 3:4]

    mmax = jnp.max(mk, axis=1, keepdims=True)
    e = jnp.exp(mk - mmax)
    s = jnp.sum(e, axis=1, keepdims=True)
    m_ref[...] = e / s

    lane = jax.lax.broadcasted_iota(jnp.int32, (x.shape[0], _I), 1)
    big = jnp.int32(2 ** 30)
    ind = jnp.min(jnp.where(mk == mmax, lane, big), axis=1)
    ind_ref[0, 0, :] = ind

    onehot = ind[:, None] == lane

    @pl.when(i == 0)
    def _init():
        smax_ref[...] = jnp.full(smax_ref.shape, _NEG, jnp.float32)
        smin_ref[...] = jnp.full(smin_ref.shape, _POS, jnp.float32)

    for c in range(3):
        col = x[:, c][:, None]
        mx = jnp.max(jnp.where(onehot, col, _NEG), axis=0)
        mn = jnp.min(jnp.where(onehot, col, _POS), axis=0)
        smax_ref[c, 0:_I] = jnp.maximum(smax_ref[c, 0:_I], mx)
        smin_ref[c, 0:_I] = jnp.minimum(smin_ref[c, 0:_I], mn)

    @pl.when(i == nb - 1)
    def _emit():
        vmax_ref[...] = smax_ref[...]
        vmin_ref[...] = smin_ref[...]


def _stage2_body(pc_ref, ind_ref, t_ref, yaw_ref, vmax_ref, vmin_ref, flow_ref):
    x = pc_ref[...]
    ind = ind_ref[0, 0, :]
    t = t_ref[...]
    z = yaw_ref[...]
    vmax = vmax_ref[...]
    vmin = vmin_ref[...]
    xc = jnp.where(vmax >= vmin, (vmax + vmin) * 0.5, 0.0)

    lane = jax.lax.broadcasted_iota(jnp.int32, (x.shape[0], _I), 1)
    onehot = ind[:, None] == lane

    cols = []
    for c in range(3):
        row = xc[c, 0:_I]
        cols.append(jnp.sum(jnp.where(onehot, row[None, :], 0.0),
                            axis=1, keepdims=True))
    pxc = jnp.concatenate(cols, axis=1)

    d = x - pxc
    a2 = z * z
    a = jnp.sqrt(a2 + 1e-12)
    small = a < 1e-4
    a_safe = jnp.where(small, 1.0, a)
    sin_term = jnp.where(small, 1.0 - a2 / 6.0, jnp.sin(a_safe) / a_safe)
    cos_term = jnp.where(small, 0.5 - a2 / 24.0,
                         (1.0 - jnp.cos(a_safe)) / (a_safe * a_safe))
    s = sin_term * z
    cm = 1.0 - cos_term * (z * z)

    dx = d[:, 0:1]
    dy = d[:, 1:2]
    dz = d[:, 2:3]
    rx = cm * dx - s * dy
    ry = s * dx + cm * dy
    rot = jnp.concatenate([rx, ry, dz], axis=1)
    flow_ref[...] = (rot + pxc + t) - x


def kernel(pc1, mask, W1, b1, W2, b2, W3, b3):
    pc = pc1.reshape(_N, 3)
    mk = mask.reshape(_N, _I)
    nb = _N // _BN
    grid = (nb,)

    m, t, yaw, ind, vmax, vmin = pl.pallas_call(
        _stage1_body,
        grid=grid,
        in_specs=[
            pl.BlockSpec((_BN, 3), lambda i: (i, 0)),
            pl.BlockSpec((_BN, _I), lambda i: (i, 0)),
            pl.BlockSpec((3, _H), lambda i: (0, 0)),
            pl.BlockSpec((_H,), lambda i: (0,)),
            pl.BlockSpec((_H, _H), lambda i: (0, 0)),
            pl.BlockSpec((_H,), lambda i: (0,)),
            pl.BlockSpec((_H, 4), lambda i: (0, 0)),
            pl.BlockSpec((4,), lambda i: (0,)),
        ],
        out_specs=[
            pl.BlockSpec((_BN, _I), lambda i: (i, 0)),
            pl.BlockSpec((_BN, 3), lambda i: (i, 0)),
            pl.BlockSpec((_BN, 1), lambda i: (i, 0)),
            pl.BlockSpec((1, 1, _BN), lambda i: (i, 0, 0)),
            pl.BlockSpec((8, 128), lambda i: (0, 0)),
            pl.BlockSpec((8, 128), lambda i: (0, 0)),
        ],
        out_shape=[
            jax.ShapeDtypeStruct((_N, _I), jnp.float32),
            jax.ShapeDtypeStruct((_N, 3), jnp.float32),
            jax.ShapeDtypeStruct((_N, 1), jnp.float32),
            jax.ShapeDtypeStruct((nb, 1, _BN), jnp.int32),
            jax.ShapeDtypeStruct((8, 128), jnp.float32),
            jax.ShapeDtypeStruct((8, 128), jnp.float32),
        ],
        scratch_shapes=[
            pltpu.VMEM((8, 128), jnp.float32),
            pltpu.VMEM((8, 128), jnp.float32),
        ],
        interpret=_INTERPRET,
    )(pc, mk, W1, b1, W2, b2, W3, b3)

    flow = pl.pallas_call(
        _stage2_body,
        grid=grid,
        in_specs=[
            pl.BlockSpec((_BN, 3), lambda i: (i, 0)),
            pl.BlockSpec((1, 1, _BN), lambda i: (i, 0, 0)),
            pl.BlockSpec((_BN, 3), lambda i: (i, 0)),
            pl.BlockSpec((_BN, 1), lambda i: (i, 0)),
            pl.BlockSpec((8, 128), lambda i: (0, 0)),
            pl.BlockSpec((8, 128), lambda i: (0, 0)),
        ],
        out_specs=pl.BlockSpec((_BN, 3), lambda i: (i, 0)),
        out_shape=jax.ShapeDtypeStruct((_N, 3), jnp.float32),
        interpret=_INTERPRET,
    )(pc, ind, t, yaw, vmax, vmin)

    return (flow.reshape(1, _N, 3), m.reshape(1, _N, _I),
            t.reshape(1, _N, 3), yaw.reshape(1, _N, 1))


# speedup vs baseline: 2.3861x; 1.4871x over previous
import jax
import jax.numpy as jnp
from jax.experimental import pallas as pl
from jax.experimental.pallas import tpu as pltpu

_N = 100000
_I = 10
_H = 128
_BN = 2000
_NEG = -3.0e38
_POS = 3.0e38
_INTERPRET = False


def _stage1_body(pc_ref, mk_ref, w1_ref, b1_ref, w2_ref, b2_ref, w3_ref, b3_ref,
                 m_ref, t_ref, yaw_ref, ind_ref, taux_ref, vmax_ref, vmin_ref,
                 smax_ref, smin_ref):
    i = pl.program_id(0)
    nb = pl.num_programs(0)
    x = pc_ref[...]
    mk = mk_ref[...]

    h = jnp.maximum(
        jnp.dot(x, w1_ref[...], preferred_element_type=jnp.float32)
        + b1_ref[...][None, :], 0.0)
    h = jnp.maximum(
        jnp.dot(h, w2_ref[...], preferred_element_type=jnp.float32)
        + b2_ref[...][None, :], 0.0)
    out4 = (jnp.dot(h, w3_ref[...], preferred_element_type=jnp.float32)
            + b3_ref[...][None, :])
    t_ref[...] = out4[:, :3]
    yaw_ref[...] = out4[:, 3:4]
    taux_ref[0] = jnp.transpose(out4)

    mmax = jnp.max(mk, axis=1, keepdims=True)
    sh = mk - mmax
    e = jnp.exp(sh)
    s = jnp.sum(e, axis=1, keepdims=True)
    m_ref[...] = jnp.exp(sh - jnp.log(s))

    lane = jax.lax.broadcasted_iota(jnp.int32, (x.shape[0], _I), 1)
    big = jnp.int32(2 ** 30)
    ind = jnp.min(jnp.where(mk == mmax, lane, big), axis=1)
    ind_ref[0, 0, :] = ind

    onehot = ind[:, None] == lane

    @pl.when(i == 0)
    def _init():
        smax_ref[...] = jnp.full(smax_ref.shape, _NEG, jnp.float32)
        smin_ref[...] = jnp.full(smin_ref.shape, _POS, jnp.float32)

    for c in range(3):
        col = x[:, c][:, None]
        mx = jnp.max(jnp.where(onehot, col, _NEG), axis=0)
        mn = jnp.min(jnp.where(onehot, col, _POS), axis=0)
        smax_ref[c, 0:_I] = jnp.maximum(smax_ref[c, 0:_I], mx)
        smin_ref[c, 0:_I] = jnp.minimum(smin_ref[c, 0:_I], mn)

    @pl.when(i == nb - 1)
    def _emit():
        vmax_ref[...] = smax_ref[...]
        vmin_ref[...] = smin_ref[...]


def _stage2_body(pc_ref, ind_ref, taux_ref, vmax_ref, vmin_ref, flow_ref):
    xT = jnp.transpose(pc_ref[...])
    ind = ind_ref[0, 0, :][None, :]
    ta = taux_ref[0]
    z = ta[3:4, :]

    zero = jnp.zeros_like(z)
    accx, accy, accz = zero, zero, zero

    def _center(c, i):
        mx = vmax_ref[c, i]
        mn = vmin_ref[c, i]
        return jnp.where(mx >= mn, (mx + mn) * 0.5, 0.0)

    for i in range(_I):
        cond = ind == i
        accx = jnp.where(cond, _center(0, i), accx)
        accy = jnp.where(cond, _center(1, i), accy)
        accz = jnp.where(cond, _center(2, i), accz)

    px = xT[0:1, :]
    py = xT[1:2, :]
    pz = xT[2:3, :]
    dx = px - accx
    dy = py - accy
    dz = pz - accz

    a2 = z * z
    a = jnp.sqrt(a2 + 1e-12)
    small = a < 1e-4
    a_safe = jnp.where(small, 1.0, a)
    sin_term = jnp.where(small, 1.0 - a2 / 6.0, jnp.sin(a_safe) / a_safe)
    cos_term = jnp.where(small, 0.5 - a2 / 24.0,
                         (1.0 - jnp.cos(a_safe)) / (a_safe * a_safe))
    s = sin_term * z
    cm = 1.0 - cos_term * a2

    rx = cm * dx - s * dy
    ry = s * dx + cm * dy
    fx = (rx + accx + ta[0:1, :]) - px
    fy = (ry + accy + ta[1:2, :]) - py
    fz = (dz + accz + ta[2:3, :]) - pz
    flowT = jnp.concatenate([fx, fy, fz], axis=0)
    flow_ref[...] = jnp.transpose(flowT)


def kernel(pc1, mask, W1, b1, W2, b2, W3, b3):
    pc = pc1.reshape(_N, 3)
    mk = mask.reshape(_N, _I)
    nb = _N // _BN
    grid = (nb,)

    m, t, yaw, ind, taux, vmax, vmin = pl.pallas_call(
        _stage1_body,
        grid=grid,
        in_specs=[
            pl.BlockSpec((_BN, 3), lambda i: (i, 0)),
            pl.BlockSpec((_BN, _I), lambda i: (i, 0)),
            pl.BlockSpec((3, _H), lambda i: (0, 0)),
            pl.BlockSpec((_H,), lambda i: (0,)),
            pl.BlockSpec((_H, _H), lambda i: (0, 0)),
            pl.BlockSpec((_H,), lambda i: (0,)),
            pl.BlockSpec((_H, 4), lambda i: (0, 0)),
            pl.BlockSpec((4,), lambda i: (0,)),
        ],
        out_specs=[
            pl.BlockSpec((_BN, _I), lambda i: (i, 0)),
            pl.BlockSpec((_BN, 3), lambda i: (i, 0)),
            pl.BlockSpec((_BN, 1), lambda i: (i, 0)),
            pl.BlockSpec((1, 1, _BN), lambda i: (i, 0, 0)),
            pl.BlockSpec((1, 4, _BN), lambda i: (i, 0, 0)),
            pl.BlockSpec((8, 128), lambda i: (0, 0)),
            pl.BlockSpec((8, 128), lambda i: (0, 0)),
        ],
        out_shape=[
            jax.ShapeDtypeStruct((_N, _I), jnp.float32),
            jax.ShapeDtypeStruct((_N, 3), jnp.float32),
            jax.ShapeDtypeStruct((_N, 1), jnp.float32),
            jax.ShapeDtypeStruct((nb, 1, _BN), jnp.int32),
            jax.ShapeDtypeStruct((nb, 4, _BN), jnp.float32),
            jax.ShapeDtypeStruct((8, 128), jnp.float32),
            jax.ShapeDtypeStruct((8, 128), jnp.float32),
        ],
        scratch_shapes=[
            pltpu.VMEM((8, 128), jnp.float32),
            pltpu.VMEM((8, 128), jnp.float32),
        ],
        interpret=_INTERPRET,
    )(pc, mk, W1, b1, W2, b2, W3, b3)

    flow = pl.pallas_call(
        _stage2_body,
        grid=grid,
        in_specs=[
            pl.BlockSpec((_BN, 3), lambda i: (i, 0)),
            pl.BlockSpec((1, 1, _BN), lambda i: (i, 0, 0)),
            pl.BlockSpec((1, 4, _BN), lambda i: (i, 0, 0)),
            pl.BlockSpec((8, 128), lambda i: (0, 0)),
            pl.BlockSpec((8, 128), lambda i: (0, 0)),
        ],
        out_specs=pl.BlockSpec((_BN, 3), lambda i: (i, 0)),
        out_shape=jax.ShapeDtypeStruct((_N, 3), jnp.float32),
        interpret=_INTERPRET,
    )(pc, ind, taux, vmax, vmin)

    return (flow.reshape(1, _N, 3), m.reshape(1, _N, _I),
            t.reshape(1, _N, 3), yaw.reshape(1, _N, 1))


# speedup vs baseline: 3.1090x; 1.3029x over previous
import jax
import jax.numpy as jnp
from jax.experimental import pallas as pl
from jax.experimental.pallas import tpu as pltpu

_N = 100000
_I = 10
_H = 128
_BN = 2000
_NEG = -3.0e38
_POS = 3.0e38
_INTERPRET = False


def _stage1_body(pc_ref, mk_ref, w1_ref, b1_ref, w2_ref, b2_ref, w3_ref, b3_ref,
                 m_ref, t_ref, yaw_ref, aux_ref, vmax_ref, vmin_ref,
                 smax_ref, smin_ref):
    i = pl.program_id(0)
    nb = pl.num_programs(0)
    x = pc_ref[...]
    mk = mk_ref[...]

    h = jnp.maximum(
        jnp.dot(x, w1_ref[...], preferred_element_type=jnp.float32)
        + b1_ref[...][None, :], 0.0)
    h = jnp.maximum(
        jnp.dot(h, w2_ref[...], preferred_element_type=jnp.float32)
        + b2_ref[...][None, :], 0.0)
    out4 = (jnp.dot(h, w3_ref[...], preferred_element_type=jnp.float32)
            + b3_ref[...][None, :])
    t_ref[...] = out4[:, :3]
    yaw_ref[...] = out4[:, 3:4]

    mmax = jnp.max(mk, axis=1, keepdims=True)
    sh = mk - mmax
    e = jnp.exp(sh)
    s = jnp.sum(e, axis=1, keepdims=True)
    m_ref[...] = jnp.exp(sh - jnp.log(s))

    bn = x.shape[0]
    mkT = jnp.transpose(mk)
    pcT = jnp.transpose(x)
    out4T = jnp.transpose(out4)
    mmaxT = jnp.max(mkT, axis=0, keepdims=True)
    subi = jax.lax.broadcasted_iota(jnp.int32, (_I, bn), 0)
    big = jnp.int32(2 ** 30)
    indT = jnp.min(jnp.where(mkT == mmaxT, subi, big),
                   axis=0, keepdims=True)
    indTf = jax.lax.bitcast_convert_type(indT, jnp.float32)
    aux_ref[0] = jnp.concatenate([out4T, pcT, indTf], axis=0)

    pc8x = jnp.concatenate([pcT, jnp.full((5, bn), _NEG, jnp.float32)], axis=0)
    pc8n = jnp.concatenate([pcT, jnp.full((5, bn), _POS, jnp.float32)], axis=0)

    @pl.when(i == 0)
    def _init():
        smax_ref[...] = jnp.full(smax_ref.shape, _NEG, jnp.float32)
        smin_ref[...] = jnp.full(smin_ref.shape, _POS, jnp.float32)

    for inst in range(_I):
        cond8 = jnp.broadcast_to(indT == inst, (8, bn))
        smax_ref[inst] = jnp.maximum(smax_ref[inst],
                                     jnp.where(cond8, pc8x, _NEG))
        smin_ref[inst] = jnp.minimum(smin_ref[inst],
                                     jnp.where(cond8, pc8n, _POS))

    @pl.when(i == nb - 1)
    def _emit():
        mxs = [jnp.max(smax_ref[inst], axis=1, keepdims=True)
               for inst in range(_I)]
        mns = [jnp.min(smin_ref[inst], axis=1, keepdims=True)
               for inst in range(_I)]
        vmax_ref[0:8, 0:_I] = jnp.concatenate(mxs, axis=1)
        vmin_ref[0:8, 0:_I] = jnp.concatenate(mns, axis=1)


def _stage2_body(aux_ref, vmax_ref, vmin_ref, flow_ref):
    ax = aux_ref[0]
    z = ax[3:4, :]
    px = ax[4:5, :]
    py = ax[5:6, :]
    pz = ax[6:7, :]
    ind = jax.lax.bitcast_convert_type(ax[7:8, :], jnp.int32)

    zero = jnp.zeros_like(z)
    accx, accy, accz = zero, zero, zero

    def _center(c, i):
        mx = vmax_ref[c, i]
        mn = vmin_ref[c, i]
        return jnp.where(mx >= mn, (mx + mn) * 0.5, 0.0)

    for i in range(_I):
        cond = ind == i
        accx = jnp.where(cond, _center(0, i), accx)
        accy = jnp.where(cond, _center(1, i), accy)
        accz = jnp.where(cond, _center(2, i), accz)

    dx = px - accx
    dy = py - accy
    dz = pz - accz

    a2 = z * z
    a = jnp.sqrt(a2 + 1e-12)
    small = a < 1e-4
    a_safe = jnp.where(small, 1.0, a)
    sin_term = jnp.where(small, 1.0 - a2 / 6.0, jnp.sin(a_safe) / a_safe)
    cos_term = jnp.where(small, 0.5 - a2 / 24.0,
                         (1.0 - jnp.cos(a_safe)) / (a_safe * a_safe))
    s = sin_term * z
    cm = 1.0 - cos_term * a2

    rx = cm * dx - s * dy
    ry = s * dx + cm * dy
    fx = (rx + accx + ax[0:1, :]) - px
    fy = (ry + accy + ax[1:2, :]) - py
    fz = (dz + accz + ax[2:3, :]) - pz
    flowT = jnp.concatenate([fx, fy, fz], axis=0)
    flow_ref[...] = jnp.transpose(flowT)


def kernel(pc1, mask, W1, b1, W2, b2, W3, b3):
    pc = pc1.reshape(_N, 3)
    mk = mask.reshape(_N, _I)
    nb = _N // _BN
    grid = (nb,)

    m, t, yaw, aux, vmax, vmin = pl.pallas_call(
        _stage1_body,
        grid=grid,
        in_specs=[
            pl.BlockSpec((_BN, 3), lambda i: (i, 0)),
            pl.BlockSpec((_BN, _I), lambda i: (i, 0)),
            pl.BlockSpec((3, _H), lambda i: (0, 0)),
            pl.BlockSpec((_H,), lambda i: (0,)),
            pl.BlockSpec((_H, _H), lambda i: (0, 0)),
            pl.BlockSpec((_H,), lambda i: (0,)),
            pl.BlockSpec((_H, 4), lambda i: (0, 0)),
            pl.BlockSpec((4,), lambda i: (0,)),
        ],
        out_specs=[
            pl.BlockSpec((_BN, _I), lambda i: (i, 0)),
            pl.BlockSpec((_BN, 3), lambda i: (i, 0)),
            pl.BlockSpec((_BN, 1), lambda i: (i, 0)),
            pl.BlockSpec((1, 8, _BN), lambda i: (i, 0, 0)),
            pl.BlockSpec((8, 128), lambda i: (0, 0)),
            pl.BlockSpec((8, 128), lambda i: (0, 0)),
        ],
        out_shape=[
            jax.ShapeDtypeStruct((_N, _I), jnp.float32),
            jax.ShapeDtypeStruct((_N, 3), jnp.float32),
            jax.ShapeDtypeStruct((_N, 1), jnp.float32),
            jax.ShapeDtypeStruct((nb, 8, _BN), jnp.float32),
            jax.ShapeDtypeStruct((8, 128), jnp.float32),
            jax.ShapeDtypeStruct((8, 128), jnp.float32),
        ],
        scratch_shapes=[
            pltpu.VMEM((_I, 8, _BN), jnp.float32),
            pltpu.VMEM((_I, 8, _BN), jnp.float32),
        ],
        interpret=_INTERPRET,
    )(pc, mk, W1, b1, W2, b2, W3, b3)

    flow = pl.pallas_call(
        _stage2_body,
        grid=grid,
        in_specs=[
            pl.BlockSpec((1, 8, _BN), lambda i: (i, 0, 0)),
            pl.BlockSpec((8, 128), lambda i: (0, 0)),
            pl.BlockSpec((8, 128), lambda i: (0, 0)),
        ],
        out_specs=pl.BlockSpec((_BN, 3), lambda i: (i, 0)),
        out_shape=jax.ShapeDtypeStruct((_N, 3), jnp.float32),
        interpret=_INTERPRET,
    )(aux, vmax, vmin)

    return (flow.reshape(1, _N, 3), m.reshape(1, _N, _I),
            t.reshape(1, _N, 3), yaw.reshape(1, _N, 1))


# speedup vs baseline: 3.3919x; 1.0910x over previous
import jax
import jax.numpy as jnp
from jax.experimental import pallas as pl
from jax.experimental.pallas import tpu as pltpu

_N = 100000
_I = 10
_H = 128
_BN = 4000
_NEG = -3.0e38
_POS = 3.0e38
_INTERPRET = False


def _stage1_body(pc_ref, mk_ref, w1_ref, b1_ref, w2_ref, b2_ref, w3_ref, b3_ref,
                 m_ref, t_ref, yaw_ref, aux_ref, vmax_ref, vmin_ref,
                 smax_ref, smin_ref):
    i = pl.program_id(0)
    nb = pl.num_programs(0)
    x = pc_ref[...]
    mk = mk_ref[...]

    h = jnp.maximum(
        jnp.dot(x, w1_ref[...], preferred_element_type=jnp.float32)
        + b1_ref[...][None, :], 0.0)
    h = jnp.maximum(
        jnp.dot(h, w2_ref[...], preferred_element_type=jnp.float32)
        + b2_ref[...][None, :], 0.0)
    out4 = (jnp.dot(h, w3_ref[...], preferred_element_type=jnp.float32)
            + b3_ref[...][None, :])
    t_ref[...] = out4[:, :3]
    yaw_ref[...] = out4[:, 3:4]

    mmax = jnp.max(mk, axis=1, keepdims=True)
    sh = mk - mmax
    e = jnp.exp(sh)
    s = jnp.sum(e, axis=1, keepdims=True)
    m_ref[...] = jnp.exp(sh - jnp.log(s))

    bn = x.shape[0]
    mkT = jnp.transpose(mk)
    pcT = jnp.transpose(x)
    out4T = jnp.transpose(out4)
    mmaxT = jnp.max(mkT, axis=0, keepdims=True)
    subi = jax.lax.broadcasted_iota(jnp.int32, (_I, bn), 0)
    big = jnp.int32(2 ** 30)
    indT = jnp.min(jnp.where(mkT == mmaxT, subi, big),
                   axis=0, keepdims=True)
    indTf = jax.lax.bitcast_convert_type(indT, jnp.float32)
    aux_ref[0] = jnp.concatenate([out4T, pcT, indTf], axis=0)

    pc8x = jnp.concatenate([pcT, jnp.full((5, bn), _NEG, jnp.float32)], axis=0)
    pc8n = jnp.concatenate([pcT, jnp.full((5, bn), _POS, jnp.float32)], axis=0)

    @pl.when(i == 0)
    def _init():
        smax_ref[...] = jnp.full(smax_ref.shape, _NEG, jnp.float32)
        smin_ref[...] = jnp.full(smin_ref.shape, _POS, jnp.float32)

    for inst in range(_I):
        cond8 = jnp.broadcast_to(indT == inst, (8, bn))
        smax_ref[inst] = jnp.maximum(smax_ref[inst],
                                     jnp.where(cond8, pc8x, _NEG))
        smin_ref[inst] = jnp.minimum(smin_ref[inst],
                                     jnp.where(cond8, pc8n, _POS))

    @pl.when(i == nb - 1)
    def _emit():
        mxs = [jnp.max(smax_ref[inst], axis=1, keepdims=True)
               for inst in range(_I)]
        mns = [jnp.min(smin_ref[inst], axis=1, keepdims=True)
               for inst in range(_I)]
        vmax_ref[0:8, 0:_I] = jnp.concatenate(mxs, axis=1)
        vmin_ref[0:8, 0:_I] = jnp.concatenate(mns, axis=1)


def _stage2_body(aux_ref, vmax_ref, vmin_ref, flow_ref):
    ax = aux_ref[0]
    z = ax[3:4, :]
    px = ax[4:5, :]
    py = ax[5:6, :]
    pz = ax[6:7, :]
    ind = jax.lax.bitcast_convert_type(ax[7:8, :], jnp.int32)

    zero = jnp.zeros_like(z)
    accx, accy, accz = zero, zero, zero

    def _center(c, i):
        mx = vmax_ref[c, i]
        mn = vmin_ref[c, i]
        return jnp.where(mx >= mn, (mx + mn) * 0.5, 0.0)

    for i in range(_I):
        cond = ind == i
        accx = jnp.where(cond, _center(0, i), accx)
        accy = jnp.where(cond, _center(1, i), accy)
        accz = jnp.where(cond, _center(2, i), accz)

    dx = px - accx
    dy = py - accy
    dz = pz - accz

    a2 = z * z
    a = jnp.sqrt(a2 + 1e-12)
    small = a < 1e-4
    a_safe = jnp.where(small, 1.0, a)
    sin_term = jnp.where(small, 1.0 - a2 / 6.0, jnp.sin(a_safe) / a_safe)
    cos_term = jnp.where(small, 0.5 - a2 / 24.0,
                         (1.0 - jnp.cos(a_safe)) / (a_safe * a_safe))
    s = sin_term * z
    cm = 1.0 - cos_term * a2

    rx = cm * dx - s * dy
    ry = s * dx + cm * dy
    fx = (rx + accx + ax[0:1, :]) - px
    fy = (ry + accy + ax[1:2, :]) - py
    fz = (dz + accz + ax[2:3, :]) - pz
    flowT = jnp.concatenate([fx, fy, fz], axis=0)
    flow_ref[...] = jnp.transpose(flowT)


def kernel(pc1, mask, W1, b1, W2, b2, W3, b3):
    pc = pc1.reshape(_N, 3)
    mk = mask.reshape(_N, _I)
    nb = _N // _BN
    grid = (nb,)

    m, t, yaw, aux, vmax, vmin = pl.pallas_call(
        _stage1_body,
        grid=grid,
        in_specs=[
            pl.BlockSpec((_BN, 3), lambda i: (i, 0)),
            pl.BlockSpec((_BN, _I), lambda i: (i, 0)),
            pl.BlockSpec((3, _H), lambda i: (0, 0)),
            pl.BlockSpec((_H,), lambda i: (0,)),
            pl.BlockSpec((_H, _H), lambda i: (0, 0)),
            pl.BlockSpec((_H,), lambda i: (0,)),
            pl.BlockSpec((_H, 4), lambda i: (0, 0)),
            pl.BlockSpec((4,), lambda i: (0,)),
        ],
        out_specs=[
            pl.BlockSpec((_BN, _I), lambda i: (i, 0)),
            pl.BlockSpec((_BN, 3), lambda i: (i, 0)),
            pl.BlockSpec((_BN, 1), lambda i: (i, 0)),
            pl.BlockSpec((1, 8, _BN), lambda i: (i, 0, 0)),
            pl.BlockSpec((8, 128), lambda i: (0, 0)),
            pl.BlockSpec((8, 128), lambda i: (0, 0)),
        ],
        out_shape=[
            jax.ShapeDtypeStruct((_N, _I), jnp.float32),
            jax.ShapeDtypeStruct((_N, 3), jnp.float32),
            jax.ShapeDtypeStruct((_N, 1), jnp.float32),
            jax.ShapeDtypeStruct((nb, 8, _BN), jnp.float32),
            jax.ShapeDtypeStruct((8, 128), jnp.float32),
            jax.ShapeDtypeStruct((8, 128), jnp.float32),
        ],
        scratch_shapes=[
            pltpu.VMEM((_I, 8, _BN), jnp.float32),
            pltpu.VMEM((_I, 8, _BN), jnp.float32),
        ],
        interpret=_INTERPRET,
    )(pc, mk, W1, b1, W2, b2, W3, b3)

    flow = pl.pallas_call(
        _stage2_body,
        grid=grid,
        in_specs=[
            pl.BlockSpec((1, 8, _BN), lambda i: (i, 0, 0)),
            pl.BlockSpec((8, 128), lambda i: (0, 0)),
            pl.BlockSpec((8, 128), lambda i: (0, 0)),
        ],
        out_specs=pl.BlockSpec((_BN, 3), lambda i: (i, 0)),
        out_shape=jax.ShapeDtypeStruct((_N, 3), jnp.float32),
        interpret=_INTERPRET,
    )(aux, vmax, vmin)

    return (flow.reshape(1, _N, 3), m.reshape(1, _N, _I),
            t.reshape(1, _N, 3), yaw.reshape(1, _N, 1))
